# baseline (speedup 1.0000x reference)
import functools

import jax
import jax.numpy as jnp
from jax import lax
from jax.experimental import pallas as pl
from jax.experimental.pallas import tpu as pltpu
from jax.experimental.pallas import tpu_sc as plsc

_D = 1024
_F = 4096
_FH = _F // 2
_E = 8
_K = 2
_T = 4096
_A = _T * _K
_B = 128
_NB = _A // _B + _E
_NP = _NB * _B
_TB = 512
_NC = 2
_NS = 16
_NW = _NC * _NS
_L = 16


def _gate_body(x_ref, wg_ref, bg_ref, g_ref, s_ref, xbf_ref):
    xb = x_ref[...]
    logits = jnp.dot(xb, wg_ref[...],
                     preferred_element_type=jnp.float32) + bg_ref[...]
    iota = lax.broadcasted_iota(jnp.int32, logits.shape, 1)
    m1 = jnp.max(logits, axis=1, keepdims=True)
    i1 = jnp.min(jnp.where(logits == m1, iota, _E), axis=1, keepdims=True)
    sel1 = iota == i1
    neg = jnp.float32(float("-inf"))
    l2 = jnp.where(sel1, neg, logits)
    m2 = jnp.max(l2, axis=1, keepdims=True)
    i2 = jnp.min(jnp.where(l2 == m2, iota, _E), axis=1, keepdims=True)
    sel2 = iota == i2
    e21 = jnp.exp(m2 - m1)
    w1 = 1.0 / (1.0 + e21)
    w2 = e21 / (1.0 + e21)
    g_ref[...] = jnp.where(sel1, w1, 0.0) + jnp.where(sel2, w2, 0.0)
    s_ref[...] = (sel1 | sel2).astype(jnp.int32)
    xbf_ref[...] = xb.astype(jnp.bfloat16)


def _gate(x_flat, Wg, bg2d):
    return pl.pallas_call(
        _gate_body,
        grid=(_T // _TB,),
        in_specs=[
            pl.BlockSpec((_TB, _D), lambda i: (i, 0)),
            pl.BlockSpec((_D, _E), lambda i: (0, 0)),
            pl.BlockSpec((1, _E), lambda i: (0, 0)),
        ],
        out_specs=[
            pl.BlockSpec((_TB, _E), lambda i: (i, 0)),
            pl.BlockSpec((_TB, _E), lambda i: (i, 0)),
            pl.BlockSpec((_TB, _D), lambda i: (i, 0)),
        ],
        out_shape=[
            jax.ShapeDtypeStruct((_T, _E), jnp.float32),
            jax.ShapeDtypeStruct((_T, _E), jnp.int32),
            jax.ShapeDtypeStruct((_T, _D), jnp.bfloat16),
        ],
    )(x_flat, Wg, bg2d)


def _ffn_body(be_ref, bv_ref, xs_ref, w1_ref, b1_ref, w2_ref, b2_ref, wc_ref,
              ys_ref):
    @pl.when(bv_ref[pl.program_id(0)] == 1)
    def _compute():
        h = jnp.maximum(
            jnp.dot(xs_ref[...], w1_ref[0],
                    preferred_element_type=jnp.float32) + b1_ref[0], 0.0)
        hb = h.astype(jnp.bfloat16)
        out = jnp.dot(hb, w2_ref[0],
                      preferred_element_type=jnp.float32) + b2_ref[0]
        ys_ref[...] = out * wc_ref[...][:, 0:1]


def _ffn(block_expert, block_valid, xs_bf, W1bf, b1r, W2bf, b2r, w_mat):
    grid_spec = pltpu.PrefetchScalarGridSpec(
        num_scalar_prefetch=2,
        grid=(_NB,),
        in_specs=[
            pl.BlockSpec((_B, _D), lambda b, be, bv: (b, 0)),
            pl.BlockSpec((1, _D, _F), lambda b, be, bv: (be[b], 0, 0)),
            pl.BlockSpec((1, 1, _F), lambda b, be, bv: (be[b], 0, 0)),
            pl.BlockSpec((1, _F, _D), lambda b, be, bv: (be[b], 0, 0)),
            pl.BlockSpec((1, 1, _D), lambda b, be, bv: (be[b], 0, 0)),
            pl.BlockSpec((_B, 128), lambda b, be, bv: (b, 0)),
        ],
        out_specs=pl.BlockSpec((_B, _D), lambda b, be, bv: (b, 0)),
    )
    return pl.pallas_call(
        _ffn_body,
        grid_spec=grid_spec,
        out_shape=jax.ShapeDtypeStruct((_NP, _D), jnp.float32),
    )(block_expert, block_valid, xs_bf, W1bf, b1r, W2bf, b2r, w_mat)


def _sc_gather(x_i32, src_tok):
    rpw = _NP // _NW
    cs = 96
    nck = rpw // cs
    dh = _D // 2
    mesh = plsc.VectorSubcoreMesh(core_axis_name="c", subcore_axis_name="s")

    @functools.partial(
        pl.kernel, mesh=mesh,
        out_type=jax.ShapeDtypeStruct((_NP, dh), jnp.int32),
        scratch_types=[
            pltpu.VMEM((cs,), jnp.int32),
            pltpu.VMEM((cs,), jnp.int32),
            pltpu.VMEM((cs, dh), jnp.int32),
            pltpu.VMEM((cs, dh), jnp.int32),
            pltpu.SemaphoreType.DMA,
            pltpu.SemaphoreType.DMA,
            pltpu.SemaphoreType.DMA,
            pltpu.SemaphoreType.DMA,
        ],
    )
    def k(x_hbm, idx_hbm, out_hbm, idx0, idx1, rows0, rows1,
          gs0, gs1, os0, os1):
        wid = lax.axis_index("s") * _NC + lax.axis_index("c")
        base = wid * rpw
        idxs = (idx0, idx1)
        rows = (rows0, rows1)
        gsem = (gs0, gs1)
        osem = (os0, os1)

        pltpu.sync_copy(idx_hbm.at[pl.ds(base, cs)], idx0)
        gathers = [pltpu.async_copy(x_hbm.at[idx0], rows0, gs0)]
        outs = [None, None]
        for i in range(nck):
            s = i % 2
            if i + 1 < nck:
                ns = (i + 1) % 2
                pltpu.sync_copy(
                    idx_hbm.at[pl.ds(base + (i + 1) * cs, cs)], idxs[ns])
                if outs[ns] is not None:
                    outs[ns].wait()
                gathers.append(
                    pltpu.async_copy(x_hbm.at[idxs[ns]], rows[ns], gsem[ns]))
            gathers[i].wait()
            outs[s] = pltpu.async_copy(
                rows[s], out_hbm.at[pl.ds(base + i * cs, cs)], osem[s])
        for o in outs:
            if o is not None:
                o.wait()

    return k(x_i32, src_tok)


def _sc_combine(ys_flat, q2):
    tpw = _T // _NW
    cs = 16
    nck = tpw // cs
    mesh = plsc.VectorSubcoreMesh(core_axis_name="c", subcore_axis_name="s")

    @functools.partial(
        pl.kernel, mesh=mesh,
        out_type=jax.ShapeDtypeStruct((_T, _D), jnp.float32),
        scratch_types=[
            pltpu.VMEM((2 * cs,), jnp.int32),
            pltpu.VMEM((2 * cs,), jnp.int32),
            pltpu.VMEM((2 * cs, _D), jnp.float32),
            pltpu.VMEM((2 * cs, _D), jnp.float32),
            pltpu.VMEM((cs, _D), jnp.float32),
            pltpu.VMEM((cs, _D), jnp.float32),
            pltpu.SemaphoreType.DMA,
            pltpu.SemaphoreType.DMA,
            pltpu.SemaphoreType.DMA,
            pltpu.SemaphoreType.DMA,
        ],
    )
    def k(ys_hbm, q_hbm, out_hbm, i0, i1, r0, r1, o0, o1,
          gs0, gs1, os0, os1):
        wid = lax.axis_index("s") * _NC + lax.axis_index("c")
        base = wid * tpw
        idxs = (i0, i1)
        rows = (r0, r1)
        obuf = (o0, o1)
        gsem = (gs0, gs1)
        osem = (os0, os1)

        def pair(i, carry):
            c0 = 2 * i
            gathers = []
            for s in range(2):
                start = base + (c0 + s) * cs
                pltpu.sync_copy(q_hbm.at[pl.ds(2 * start, 2 * cs)], idxs[s])
                gathers.append(
                    pltpu.async_copy(ys_hbm.at[idxs[s]], rows[s], gsem[s]))
            outs = []
            for s in range(2):
                start = base + (c0 + s) * cs
                gathers[s].wait()
                r = rows[s]
                o = obuf[s]

                def radd(t, c2, r=r, o=o):
                    for j in range(_D // _L):
                        sl = pl.ds(j * _L, _L)
                        o[t, sl] = r[2 * t, sl] + r[2 * t + 1, sl]
                    return c2

                lax.fori_loop(0, cs, radd, 0)
                outs.append(pltpu.async_copy(
                    o, out_hbm.at[pl.ds(start, cs)], osem[s]))
            for ocp in outs:
                ocp.wait()
            return carry

        lax.fori_loop(0, nck // 2, pair, 0)

    return k(ys_flat, q2)


def _dispatch_metadata(G, S):
    a_idx = jnp.nonzero(S.reshape(-1) != 0, size=_A, fill_value=0)[0]
    a_idx = a_idx.astype(jnp.int32)
    tok = a_idx // _E
    expert = a_idx % _E
    wgt = G.reshape(-1)[a_idx]
    order = jnp.argsort(expert).astype(jnp.int32)
    tok_s = tok[order]
    wgt_s = wgt[order]
    exp_s = expert[order]
    counts = jnp.bincount(expert, length=_E).astype(jnp.int32)
    nb_e = (counts + _B - 1) // _B
    pad_start = (jnp.concatenate([jnp.zeros(1, jnp.int32),
                                  jnp.cumsum(nb_e)])[:_E] * _B)
    offs = jnp.concatenate([jnp.zeros(1, jnp.int32),
                            jnp.cumsum(counts)])[:_E]
    rank = jnp.arange(_A, dtype=jnp.int32) - offs[exp_s]
    dst = (pad_start[exp_s] + rank).astype(jnp.int32)
    src_tok = jnp.zeros((_NP,), jnp.int32).at[dst].set(tok_s)
    w_row = jnp.zeros((_NP,), jnp.float32).at[dst].set(wgt_s)
    blk_cum = jnp.cumsum(nb_e)
    total_blocks = blk_cum[-1]
    block_ids = jnp.arange(_NB, dtype=jnp.int32)
    block_expert = jnp.searchsorted(blk_cum, block_ids, side="right")
    block_expert = jnp.clip(block_expert, 0, _E - 1).astype(jnp.int32)
    block_valid = (block_ids < total_blocks).astype(jnp.int32)
    pos = jnp.zeros((_A,), jnp.int32).at[order].set(dst)
    posk = pos.reshape(_T, _K)
    return src_tok, w_row, block_expert, block_valid, posk[:, 0], posk[:, 1]


def kernel(x, Wg, bg, W1, b1, W2, b2):
    x_flat = x.reshape(_T, _D)
    G, S, xbf = _gate(x_flat, Wg, bg.reshape(1, _E))
    src_tok, w_row, block_expert, block_valid, p0, p1 = _dispatch_metadata(G, S)
    x_i32 = lax.bitcast_convert_type(
        xbf.reshape(_T, _D // 2, 2), jnp.int32)
    xs_i32 = _sc_gather(x_i32, src_tok)
    xs_bf = lax.bitcast_convert_type(xs_i32, jnp.bfloat16).reshape(_NP, _D)
    w_mat = jnp.broadcast_to(w_row[:, None], (_NP, 128))
    ys = _ffn(block_expert, block_valid, xs_bf, W1.astype(jnp.bfloat16),
              b1.reshape(_E, 1, _F), W2.astype(jnp.bfloat16),
              b2.reshape(_E, 1, _D), w_mat)
    q2 = jnp.stack([p0, p1], axis=1).reshape(-1)
    out = _sc_combine(ys, q2)
    return out.reshape(x.shape)


# speedup vs baseline: 1.1968x; 1.1968x over previous
import functools

import jax
import jax.numpy as jnp
from jax import lax
from jax.experimental import pallas as pl
from jax.experimental.pallas import tpu as pltpu
from jax.experimental.pallas import tpu_sc as plsc

_D = 1024
_F = 4096
_FH = _F // 2
_E = 8
_K = 2
_T = 4096
_A = _T * _K
_B = 128
_NB = _A // _B + _E
_NP = _NB * _B
_TB = 512
_NC = 2
_NS = 16
_NW = _NC * _NS
_L = 16


def _gate_body(x_ref, wg_ref, bg_ref, s1_ref, s2_ref, w1_ref, w2_ref,
               xbf_ref):
    xb = x_ref[...]
    logits = jnp.dot(xb, wg_ref[...],
                     preferred_element_type=jnp.float32) + bg_ref[...]
    iota = lax.broadcasted_iota(jnp.int32, logits.shape, 1)
    m1 = jnp.max(logits, axis=1, keepdims=True)
    i1 = jnp.min(jnp.where(logits == m1, iota, _E), axis=1, keepdims=True)
    sel1 = iota == i1
    neg = jnp.float32(float("-inf"))
    l2 = jnp.where(sel1, neg, logits)
    m2 = jnp.max(l2, axis=1, keepdims=True)
    i2 = jnp.min(jnp.where(l2 == m2, iota, _E), axis=1, keepdims=True)
    sel2 = iota == i2
    e21 = jnp.exp(m2 - m1)
    s1_ref[...] = sel1.astype(jnp.int32)
    s2_ref[...] = sel2.astype(jnp.int32)
    w1_ref[...] = 1.0 / (1.0 + e21)
    w2_ref[...] = e21 / (1.0 + e21)
    xbf_ref[...] = xb.astype(jnp.bfloat16)


def _gate(x_flat, Wg, bg2d):
    return pl.pallas_call(
        _gate_body,
        grid=(_T // _TB,),
        in_specs=[
            pl.BlockSpec((_TB, _D), lambda i: (i, 0)),
            pl.BlockSpec((_D, _E), lambda i: (0, 0)),
            pl.BlockSpec((1, _E), lambda i: (0, 0)),
        ],
        out_specs=[
            pl.BlockSpec((_TB, _E), lambda i: (i, 0)),
            pl.BlockSpec((_TB, _E), lambda i: (i, 0)),
            pl.BlockSpec((_TB, 1), lambda i: (i, 0)),
            pl.BlockSpec((_TB, 1), lambda i: (i, 0)),
            pl.BlockSpec((_TB, _D), lambda i: (i, 0)),
        ],
        out_shape=[
            jax.ShapeDtypeStruct((_T, _E), jnp.int32),
            jax.ShapeDtypeStruct((_T, _E), jnp.int32),
            jax.ShapeDtypeStruct((_T, 1), jnp.float32),
            jax.ShapeDtypeStruct((_T, 1), jnp.float32),
            jax.ShapeDtypeStruct((_T, _D), jnp.bfloat16),
        ],
    )(x_flat, Wg, bg2d)


def _ffn_body(be_ref, bv_ref, xs_ref, w1_ref, b1_ref, w2_ref, b2_ref, wc_ref,
              ys_ref):
    @pl.when(bv_ref[pl.program_id(0)] == 1)
    def _compute():
        h = jnp.maximum(
            jnp.dot(xs_ref[...], w1_ref[0],
                    preferred_element_type=jnp.float32) + b1_ref[0], 0.0)
        hb = h.astype(jnp.bfloat16)
        out = jnp.dot(hb, w2_ref[0],
                      preferred_element_type=jnp.float32) + b2_ref[0]
        ys_ref[...] = out * wc_ref[...][:, 0:1]


def _ffn(block_expert, block_valid, xs_bf, W1bf, b1r, W2bf, b2r, w_mat):
    grid_spec = pltpu.PrefetchScalarGridSpec(
        num_scalar_prefetch=2,
        grid=(_NB,),
        in_specs=[
            pl.BlockSpec((_B, _D), lambda b, be, bv: (b, 0)),
            pl.BlockSpec((1, _D, _F), lambda b, be, bv: (be[b], 0, 0)),
            pl.BlockSpec((1, 1, _F), lambda b, be, bv: (be[b], 0, 0)),
            pl.BlockSpec((1, _F, _D), lambda b, be, bv: (be[b], 0, 0)),
            pl.BlockSpec((1, 1, _D), lambda b, be, bv: (be[b], 0, 0)),
            pl.BlockSpec((_B, 128), lambda b, be, bv: (b, 0)),
        ],
        out_specs=pl.BlockSpec((_B, _D), lambda b, be, bv: (b, 0)),
    )
    return pl.pallas_call(
        _ffn_body,
        grid_spec=grid_spec,
        out_shape=jax.ShapeDtypeStruct((_NP, _D), jnp.float32),
    )(block_expert, block_valid, xs_bf, W1bf, b1r, W2bf, b2r, w_mat)


def _sc_dispatch(x_i32, p0, p1, w0, w1):
    tpw = _T // _NW
    dh = _D // 2
    mesh = plsc.VectorSubcoreMesh(core_axis_name="c", subcore_axis_name="s")

    @functools.partial(
        pl.kernel, mesh=mesh,
        out_type=[
            jax.ShapeDtypeStruct((_NP, dh), jnp.int32),
            jax.ShapeDtypeStruct((_NP,), jnp.float32),
        ],
        scratch_types=[
            pltpu.VMEM((tpw, dh), jnp.int32),
            pltpu.VMEM((tpw,), jnp.int32),
            pltpu.VMEM((tpw,), jnp.int32),
            pltpu.VMEM((tpw,), jnp.float32),
            pltpu.VMEM((tpw,), jnp.float32),
            pltpu.SemaphoreType.DMA,
            pltpu.SemaphoreType.DMA,
            pltpu.SemaphoreType.DMA,
            pltpu.SemaphoreType.DMA,
        ],
    )
    def k(x_hbm, p0_hbm, p1_hbm, w0_hbm, w1_hbm, outx_hbm, outw_hbm,
          xrows, i0, i1, wb0, wb1, s0, s1, s2, s3):
        wid = lax.axis_index("s") * _NC + lax.axis_index("c")
        base = wid * tpw
        sl = pl.ds(base, tpw)
        pltpu.sync_copy(p0_hbm.at[sl], i0)
        pltpu.sync_copy(p1_hbm.at[sl], i1)
        pltpu.sync_copy(w0_hbm.at[sl], wb0)
        pltpu.sync_copy(w1_hbm.at[sl], wb1)
        pltpu.sync_copy(x_hbm.at[sl], xrows)
        c0 = pltpu.async_copy(xrows, outx_hbm.at[i0], s0)
        c1 = pltpu.async_copy(xrows, outx_hbm.at[i1], s1)
        c2 = pltpu.async_copy(wb0, outw_hbm.at[i0], s2)
        c3 = pltpu.async_copy(wb1, outw_hbm.at[i1], s3)
        c0.wait()
        c1.wait()
        c2.wait()
        c3.wait()

    return k(x_i32, p0, p1, w0, w1)


def _sc_combine(ys_flat, q2):
    tpw = _T // _NW
    cs = 16
    nck = tpw // cs
    mesh = plsc.VectorSubcoreMesh(core_axis_name="c", subcore_axis_name="s")

    @functools.partial(
        pl.kernel, mesh=mesh,
        out_type=jax.ShapeDtypeStruct((_T, _D), jnp.float32),
        scratch_types=[
            pltpu.VMEM((2 * cs,), jnp.int32),
            pltpu.VMEM((2 * cs,), jnp.int32),
            pltpu.VMEM((2 * cs, _D), jnp.float32),
            pltpu.VMEM((2 * cs, _D), jnp.float32),
            pltpu.VMEM((cs, _D), jnp.float32),
            pltpu.VMEM((cs, _D), jnp.float32),
            pltpu.SemaphoreType.DMA,
            pltpu.SemaphoreType.DMA,
            pltpu.SemaphoreType.DMA,
            pltpu.SemaphoreType.DMA,
        ],
    )
    def k(ys_hbm, q_hbm, out_hbm, i0, i1, r0, r1, o0, o1,
          gs0, gs1, os0, os1):
        wid = lax.axis_index("s") * _NC + lax.axis_index("c")
        base = wid * tpw
        idxs = (i0, i1)
        rows = (r0, r1)
        obuf = (o0, o1)
        gsem = (gs0, gs1)
        osem = (os0, os1)

        def pair(i, carry):
            c0 = 2 * i
            gathers = []
            for s in range(2):
                start = base + (c0 + s) * cs
                pltpu.sync_copy(q_hbm.at[pl.ds(2 * start, 2 * cs)], idxs[s])
                gathers.append(
                    pltpu.async_copy(ys_hbm.at[idxs[s]], rows[s], gsem[s]))
            outs = []
            for s in range(2):
                start = base + (c0 + s) * cs
                gathers[s].wait()
                r = rows[s]
                o = obuf[s]

                def radd(t, c2, r=r, o=o):
                    for j in range(_D // _L):
                        sl = pl.ds(j * _L, _L)
                        o[t, sl] = r[2 * t, sl] + r[2 * t + 1, sl]
                    return c2

                lax.fori_loop(0, cs, radd, 0)
                outs.append(pltpu.async_copy(
                    o, out_hbm.at[pl.ds(start, cs)], osem[s]))
            for ocp in outs:
                ocp.wait()
            return carry

        lax.fori_loop(0, nck // 2, pair, 0)

    return k(ys_flat, q2)


def _dispatch_metadata(s1, s2):
    S = s1 + s2
    C = jnp.cumsum(S, axis=0) - S
    counts = jnp.sum(S, axis=0)
    nb_e = (counts + _B - 1) // _B
    blk_cum = jnp.cumsum(nb_e)
    pad_start = (blk_cum - nb_e) * _B
    dst = pad_start[None, :] + C
    p0 = jnp.sum(s1 * dst, axis=1).astype(jnp.int32)
    p1 = jnp.sum(s2 * dst, axis=1).astype(jnp.int32)
    block_ids = jnp.arange(_NB, dtype=jnp.int32)
    block_expert = jnp.searchsorted(blk_cum, block_ids, side="right")
    block_expert = jnp.clip(block_expert, 0, _E - 1).astype(jnp.int32)
    block_valid = (block_ids < blk_cum[-1]).astype(jnp.int32)
    return p0, p1, block_expert, block_valid


def kernel(x, Wg, bg, W1, b1, W2, b2):
    x_flat = x.reshape(_T, _D)
    s1, s2, w0, w1, xbf = _gate(x_flat, Wg, bg.reshape(1, _E))
    p0, p1, block_expert, block_valid = _dispatch_metadata(s1, s2)
    x_i32 = lax.bitcast_convert_type(
        xbf.reshape(_T, _D // 2, 2), jnp.int32)
    xs_i32, w_row = _sc_dispatch(x_i32, p0, p1,
                                 w0.reshape(_T), w1.reshape(_T))
    xs_bf = lax.bitcast_convert_type(xs_i32, jnp.bfloat16).reshape(_NP, _D)
    w_mat = jnp.broadcast_to(w_row[:, None], (_NP, 128))
    ys = _ffn(block_expert, block_valid, xs_bf, W1.astype(jnp.bfloat16),
              b1.reshape(_E, 1, _F), W2.astype(jnp.bfloat16),
              b2.reshape(_E, 1, _D), w_mat)
    q2 = jnp.stack([p0, p1], axis=1).reshape(-1)
    out = _sc_combine(ys, q2)
    return out.reshape(x.shape)
